# baseline (speedup 1.0000x reference)
import functools

import jax
import jax.numpy as jnp
from jax import lax
from jax.experimental import pallas as pl
from jax.experimental.pallas import tpu as pltpu
from jax.experimental.pallas import tpu_sc as plsc

_NC = 2
_NS = 16
_NW = _NC * _NS
_L = 16


def kernel(xs, input_table, gt_table):
    B = xs.shape[0]
    V, D = input_table.shape
    assert B % (8 * _NW) == 0 and D % _L == 0
    b_per_w = B // _NW
    n_chunks = b_per_w // _L

    mesh = plsc.VectorSubcoreMesh(core_axis_name="c", subcore_axis_name="s")

    @functools.partial(
        pl.kernel,
        mesh=mesh,
        compiler_params=pltpu.CompilerParams(use_tc_tiling_on_sc=False),
        out_type=(
            jax.ShapeDtypeStruct((B, D), jnp.float32),
            jax.ShapeDtypeStruct((B, D), jnp.float32),
        ),
        scratch_types=[
            pltpu.VMEM((b_per_w,), jnp.float32),
            pltpu.VMEM((b_per_w,), jnp.int32),
            pltpu.VMEM((b_per_w, D), jnp.float32),
            pltpu.VMEM((b_per_w, D), jnp.float32),
            pltpu.SemaphoreType.DMA,
            pltpu.SemaphoreType.DMA,
        ],
    )
    def sc_kernel(xs_hbm, in_hbm, gt_hbm, out_in_hbm, out_gt_hbm,
                  xs_v, idx_v, rows_in, rows_gt, sem_in, sem_gt):
        wid = lax.axis_index("s") * _NC + lax.axis_index("c")
        base = wid * b_per_w

        pltpu.sync_copy(xs_hbm.at[pl.ds(base, b_per_w)], xs_v)

        scale = jnp.float32(V)
        hi = jnp.float32(V - 1)

        def body(i, carry):
            v = xs_v[pl.ds(i * _L, _L)]
            scaled = v * scale
            clipped = jnp.minimum(jnp.maximum(scaled, jnp.float32(0.0)), hi)
            idx_v[pl.ds(i * _L, _L)] = clipped.astype(jnp.int32)
            return carry

        lax.fori_loop(0, n_chunks, body, 0)

        cp_in = pltpu.async_copy(in_hbm.at[idx_v], rows_in, sem_in)
        cp_gt = pltpu.async_copy(gt_hbm.at[idx_v], rows_gt, sem_gt)
        cp_in.wait()
        cp_gt.wait()

        pltpu.sync_copy(rows_in, out_in_hbm.at[pl.ds(base, b_per_w)])
        pltpu.sync_copy(rows_gt, out_gt_hbm.at[pl.ds(base, b_per_w)])

    return sc_kernel(xs, input_table, gt_table)
